# baseline (speedup 1.0000x reference)
import jax
import jax.numpy as jnp
from jax import lax
from jax.experimental import pallas as pl
from jax.experimental.pallas import tpu as pltpu
from jax.experimental.pallas import tpu_sc as plsc

B = 4096
N_SPARSE = 26
N_DENSE = 13
VOCAB = 100000
LANES = 16

NC = 2
NS = 16
NW = NC * NS
RPW = B // NW
NSL = RPW // LANES


def _sc_body(xt_hbm, tab_hbm, w_hbm, out_hbm,
             xt_v, flat_v, rows_v, w_v, acc_v, sem):
    wid = lax.axis_index("s") * NC + lax.axis_index("c")
    base = wid * RPW
    pltpu.sync_copy(xt_hbm.at[:, pl.ds(base, RPW)], xt_v)
    pltpu.sync_copy(w_hbm, w_v)
    for f in range(N_SPARSE):
        for i in range(NSL):
            sl = pl.ds(i * LANES, LANES)
            flat_v[f, sl] = xt_v[f, sl].astype(jnp.int32) + f * VOCAB
    cps = [pltpu.async_copy(tab_hbm.at[flat_v.at[f]], rows_v.at[f], sem)
           for f in range(N_SPARSE)]
    for cp in cps:
        cp.wait()
    ws = [w_v[d, :] for d in range(N_DENSE)]
    for i in range(NSL):
        sl = pl.ds(i * LANES, LANES)
        acc = rows_v[0, sl]
        for f in range(1, N_SPARSE):
            acc = acc + rows_v[f, sl]
        for d in range(N_DENSE):
            acc = acc + xt_v[N_SPARSE + d, sl] * ws[d]
        acc_v[sl] = acc
    pltpu.sync_copy(acc_v, out_hbm.at[pl.ds(base, RPW)])


def kernel(X, tables, weight):
    xt = X.T
    tab_flat = tables.reshape(-1)
    w_rep = jnp.broadcast_to(weight, (N_DENSE, LANES))
    mesh = plsc.VectorSubcoreMesh(core_axis_name="c", subcore_axis_name="s")
    k = pl.kernel(
        _sc_body,
        out_type=jax.ShapeDtypeStruct((B,), jnp.float32),
        mesh=mesh,
        scratch_types=[
            pltpu.VMEM((N_SPARSE + N_DENSE, RPW), jnp.float32),
            pltpu.VMEM((N_SPARSE, RPW), jnp.int32),
            pltpu.VMEM((N_SPARSE, RPW), jnp.float32),
            pltpu.VMEM((N_DENSE, LANES), jnp.float32),
            pltpu.VMEM((RPW,), jnp.float32),
            pltpu.SemaphoreType.DMA,
        ],
    )
    out = k(xt, tab_flat, w_rep)
    return out.reshape(B, 1)


# speedup vs baseline: 2.1579x; 2.1579x over previous
import jax
import jax.numpy as jnp
from jax import lax
from jax.experimental import pallas as pl
from jax.experimental.pallas import tpu as pltpu
from jax.experimental.pallas import tpu_sc as plsc

B = 4096
N_SPARSE = 26
N_DENSE = 13
N_COLS = N_SPARSE + N_DENSE
VOCAB = 100000
LANES = 16

NC = 2
NS = 16
NW = NC * NS
RPW = B // NW
NSL = RPW // LANES


def _sc_body(*refs):
    xt_hbm, w_hbm = refs[0], refs[1]
    tab_hbms = refs[2:2 + N_SPARSE]
    out_hbm = refs[2 + N_SPARSE]
    xt_v, idx_v, rows_v, w_v, acc_v, sem = refs[3 + N_SPARSE:]
    wid = lax.axis_index("s") * NC + lax.axis_index("c")
    base = wid * RPW
    pltpu.sync_copy(xt_hbm.at[:, pl.ds(base, RPW)], xt_v)
    pltpu.sync_copy(w_hbm, w_v)
    for f in range(N_SPARSE):
        for i in range(NSL):
            sl = pl.ds(i * LANES, LANES)
            idx_v[f, sl] = xt_v[f, sl].astype(jnp.int32)
    cps = [pltpu.async_copy(tab_hbms[f].at[idx_v.at[f]], rows_v.at[f], sem)
           for f in range(N_SPARSE)]
    for cp in cps:
        cp.wait()
    ws = [w_v[d, :] for d in range(N_DENSE)]
    for i in range(NSL):
        sl = pl.ds(i * LANES, LANES)
        acc = rows_v[0, sl]
        for f in range(1, N_SPARSE):
            acc = acc + rows_v[f, sl]
        for d in range(N_DENSE):
            acc = acc + xt_v[N_SPARSE + d, sl] * ws[d]
        acc_v[sl] = acc
    pltpu.sync_copy(acc_v, out_hbm.at[pl.ds(base, RPW)])


def kernel(X, tables, weight):
    xt = X.T
    tabs = [tables[f, :, 0] for f in range(N_SPARSE)]
    w_rep = jnp.broadcast_to(weight, (N_DENSE, LANES))
    mesh = plsc.VectorSubcoreMesh(core_axis_name="c", subcore_axis_name="s")
    k = pl.kernel(
        _sc_body,
        out_type=jax.ShapeDtypeStruct((B,), jnp.float32),
        mesh=mesh,
        scratch_types=[
            pltpu.VMEM((N_COLS, RPW), jnp.float32),
            pltpu.VMEM((N_SPARSE, RPW), jnp.int32),
            pltpu.VMEM((N_SPARSE, RPW), jnp.float32),
            pltpu.VMEM((N_DENSE, LANES), jnp.float32),
            pltpu.VMEM((RPW,), jnp.float32),
            pltpu.SemaphoreType.DMA,
        ],
    )
    out = k(xt, w_rep, *tabs)
    return out.reshape(B, 1)
